# baseline (speedup 1.0000x reference)
import functools

import jax
import jax.numpy as jnp
from jax import lax
from jax.experimental import pallas as pl
from jax.experimental.pallas import tpu as pltpu
from jax.experimental.pallas import tpu_sc as plsc

D_MODEL = 768
SEQ = 512
N_CORES = 2
N_SUBCORES = 16
N_WORKERS = N_CORES * N_SUBCORES
I_PER_W = SEQ // N_WORKERS


BLK = SEQ * D_MODEL


def _sc_copy(emb_rev_hbm, out_hbm):
    wid = lax.axis_index("s") * N_CORES + lax.axis_index("c")
    base_i = wid * I_PER_W
    for t in range(I_PER_W):
        i = base_i + t
        pltpu.sync_copy(
            emb_rev_hbm.at[pl.ds((SEQ - i) * D_MODEL, BLK)],
            out_hbm.at[pl.ds(i * BLK, BLK)],
        )


def kernel(seq_len, emb):
    del seq_len
    emb_rev = emb[::-1].reshape(-1)
    mesh = plsc.VectorSubcoreMesh(core_axis_name="c", subcore_axis_name="s")
    out_flat = pl.kernel(
        _sc_copy,
        mesh=mesh,
        out_type=jax.ShapeDtypeStruct((SEQ * SEQ * D_MODEL,), jnp.float32),
    )(emb_rev)
    return out_flat.reshape(SEQ, SEQ, D_MODEL)
